# baseline (speedup 1.0000x reference)
import jax
import jax.numpy as jnp
from jax import lax
from jax.experimental import pallas as pl
from jax.experimental.pallas import tpu as pltpu

SMOOTH = 1e-05

N, C, H, W = 4, 21, 512, 512
BH = 512
NB = H // BH
CPAD = 24
SHIFT = 4096


def _body(x_ref, t_ref, out_ref, acc):
    i = pl.program_id(0)
    n = i // NB

    @pl.when(i == 0)
    def _init():
        acc[...] = jnp.zeros((N, CPAD, 8, 128), jnp.int32)

    t = t_ref[0]
    m = x_ref[0, 0]
    pred = jnp.zeros((BH, W), jnp.int32)
    for c in range(1, C):
        xc = x_ref[0, c]
        gt = xc > m
        pred = jnp.where(gt, c, pred)
        m = jnp.maximum(xc, m)
    enc = jnp.where(pred == t, 1 + SHIFT, 1)
    for c in range(C):
        ec = jnp.where(t == c, enc, 0)
        p = ec[0:8, :]
        for s in range(1, BH // 8):
            p = p + ec[s * 8:(s + 1) * 8, :]
        q = ((p[:, 0:128] + p[:, 128:256])
             + (p[:, 256:384] + p[:, 384:512]))
        acc[n, c] += q

    @pl.when(i == N * NB - 1)
    def _fin():
        a = acc[...]
        tp = a // SHIFT
        tot = a - tp * SHIFT
        tps = jnp.sum(tp.astype(jnp.float32), axis=(2, 3))
        tots = jnp.sum(tot.astype(jnp.float32), axis=(2, 3))
        rec = (tps + SMOOTH) / (tots + SMOOTH)
        cmask = lax.broadcasted_iota(jnp.int32, (N, CPAD), 1) < C
        s = jnp.sum(jnp.where(cmask, rec, 0.0))
        out_ref[0, 0] = 1.0 - s / (N * C)


def kernel(input, target):
    t = target.astype(jnp.int32)
    out = pl.pallas_call(
        _body,
        grid=(N * NB,),
        in_specs=[
            pl.BlockSpec((1, C, BH, W), lambda i: (i // NB, 0, i % NB, 0)),
            pl.BlockSpec((1, BH, W), lambda i: (i // NB, i % NB, 0)),
        ],
        out_specs=pl.BlockSpec(memory_space=pltpu.SMEM),
        out_shape=jax.ShapeDtypeStruct((1, 1), jnp.float32),
        scratch_shapes=[
            pltpu.VMEM((N, CPAD, 8, 128), jnp.int32),
        ],
    )(input, t)
    return out[0, 0]
